# Initial kernel scaffold; baseline (speedup 1.0000x reference)
import math

import jax
import jax.numpy as jnp
from jax.experimental import pallas as pl
from jax.experimental.pallas import tpu as pltpu

_LN10 = math.log(10.0)
_LN125 = math.log(1.25)

_ROWS = 2048
_COLS = 1024
_BLOCK_ROWS = 256


def _metrics_kernel(o_ref, t_ref, out_ref, acc_ref):
    step = pl.program_id(0)
    nsteps = pl.num_programs(0)

    o = o_ref[...]
    t = t_ref[...]
    m = t > 0.001
    o = jnp.where(m, o, 1.0)
    t = jnp.where(m, t, 1.0)
    mf = m.astype(jnp.float32)

    d = jnp.abs(o - t)
    d2 = d * d
    rt = 1.0 / t
    ro = 1.0 / o
    lr = jnp.log(o * rt)
    alr = jnp.abs(lr)
    rdiff = ro - rt
    one = jnp.float32(1.0)
    zero = jnp.float32(0.0)

    def rsum(x):
        return jnp.sum(x, axis=0, keepdims=True)

    partials = jnp.concatenate(
        [
            rsum(mf),
            rsum(d2),
            rsum(d),
            rsum(lr * lr),
            rsum(alr),
            rsum(d * rt),
            rsum(d2 * rt),
            rsum(jnp.where(m & (alr < _LN125), one, zero)),
            rsum(jnp.where(m & (alr < 2.0 * _LN125), one, zero)),
            rsum(jnp.where(m & (alr < 3.0 * _LN125), one, zero)),
            rsum(rdiff * rdiff),
            rsum(jnp.abs(rdiff)),
        ],
        axis=0,
    )

    @pl.when(step == 0)
    def _init():
        acc_ref[...] = partials

    @pl.when(step != 0)
    def _accum():
        acc_ref[...] += partials

    @pl.when(step == nsteps - 1)
    def _finalize():
        s = jnp.sum(acc_ref[...], axis=1)
        inv_count = 1.0 / s[0]
        rmse = jnp.sqrt(s[1] * inv_count)
        mae = s[2] * inv_count
        rmse_log = jnp.sqrt(s[3] * inv_count)
        lg10 = s[4] * inv_count / _LN10
        absrel = s[5] * inv_count
        squared_rel = s[6] * inv_count
        delta1 = s[7] * inv_count
        delta2 = s[8] * inv_count
        delta3 = s[9] * inv_count
        irmse = 1000.0 * jnp.sqrt(s[10] * inv_count)
        imae = 1000.0 * s[11] * inv_count
        out_ref[0] = rmse
        out_ref[1] = mae
        out_ref[2] = absrel
        out_ref[3] = delta1
        out_ref[4] = delta2
        out_ref[5] = delta3
        out_ref[6] = irmse
        out_ref[7] = imae
        out_ref[8] = squared_rel
        out_ref[9] = lg10


# speedup vs baseline: 1.1603x; 1.1603x over previous
import math

import jax
import jax.numpy as jnp
from jax.experimental import pallas as pl
from jax.experimental.pallas import tpu as pltpu

_LN125 = math.log(1.25)

_ROWS = 2048
_COLS = 1024
_BLOCK_ROWS = 256


def _metrics_kernel(o_ref, t_ref, out_ref, acc_ref):
    step = pl.program_id(0)
    nsteps = pl.num_programs(0)

    o = o_ref[...]
    t = t_ref[...]
    m = t > 0.001
    o = jnp.where(m, o, 1.0)
    t = jnp.where(m, t, 1.0)
    mf = m.astype(jnp.float32)

    d = jnp.abs(o - t)
    d2 = d * d
    rt = 1.0 / t
    ro = 1.0 / o
    lr = jnp.log(o * rt)
    alr = jnp.abs(lr)
    rdiff = ro - rt
    one = jnp.float32(1.0)
    zero = jnp.float32(0.0)

    def rsum(x):
        return jnp.sum(x, axis=0, keepdims=True)

    partials = jnp.concatenate(
        [
            rsum(mf),
            rsum(d2),
            rsum(d),
            rsum(lr * lr),
            rsum(d * rt),
            rsum(d2 * rt),
            rsum(jnp.where(m & (alr < _LN125), one, zero)),
            rsum(jnp.where(m & (alr < 2.0 * _LN125), one, zero)),
            rsum(jnp.where(m & (alr < 3.0 * _LN125), one, zero)),
            rsum(rdiff * rdiff),
            rsum(jnp.abs(rdiff)),
        ],
        axis=0,
    )

    @pl.when(step == 0)
    def _init():
        acc_ref[...] = partials

    @pl.when(step != 0)
    def _accum():
        acc_ref[...] += partials

    @pl.when(step == nsteps - 1)
    def _finalize():
        s = jnp.sum(acc_ref[...], axis=1)
        inv_count = 1.0 / s[0]
        out_ref[0] = jnp.sqrt(s[1] * inv_count)
        out_ref[1] = s[2] * inv_count
        out_ref[2] = s[4] * inv_count
        out_ref[3] = s[6] * inv_count
        out_ref[4] = s[7] * inv_count
        out_ref[5] = s[8] * inv_count
        out_ref[6] = 1000.0 * jnp.sqrt(s[9] * inv_count)
        out_ref[7] = 1000.0 * s[10] * inv_count
        out_ref[8] = s[5] * inv_count
        out_ref[9] = jnp.sqrt(s[3] * inv_count)


def kernel(outputs, target):
    o = outputs.reshape(_ROWS, _COLS)
    t = target.reshape(_ROWS, _COLS)
    grid = _ROWS // _BLOCK_ROWS
    res = pl.pallas_call(
        _metrics_kernel,
        grid=(grid,),
        in_specs=[
            pl.BlockSpec((_BLOCK_ROWS, _COLS), lambda i: (i, 0)),
            pl.BlockSpec((_BLOCK_ROWS, _COLS), lambda i: (i, 0)),
        ],
        out_specs=pl.BlockSpec(memory_space=pltpu.SMEM),
        out_shape=jax.ShapeDtypeStruct((10,), jnp.float32),
        scratch_shapes=[pltpu.VMEM((11, _COLS), jnp.float32)],
    )(o, t)
    return (res[0], res[1], res[2], res[3], res[4], res[5], res[6], res[7],
            res[8], res[9])


# speedup vs baseline: 1.3592x; 1.1714x over previous
import math

import jax
import jax.numpy as jnp
from jax.experimental import pallas as pl
from jax.experimental.pallas import tpu as pltpu

_LN125 = math.log(1.25)

_ROWS = 2048
_COLS = 1024
_BLOCK_ROWS = 512
_CHUNK_ROWS = 8
_NQ = 11


def _lane_reduce(q):
    parts = [q[:, j * 128:(j + 1) * 128] for j in range(_COLS // 128)]
    while len(parts) > 1:
        parts = [a + b for a, b in zip(parts[::2], parts[1::2])]
    return parts[0]


def _metrics_kernel(o_ref, t_ref, out_ref, acc_ref):
    step = pl.program_id(0)
    nsteps = pl.num_programs(0)
    big = jnp.float32(1e30)
    one = jnp.float32(1.0)
    zero = jnp.float32(0.0)

    def body(i, carry):
        o = o_ref[pl.ds(i * _CHUNK_ROWS, _CHUNK_ROWS), :]
        t_raw = t_ref[pl.ds(i * _CHUNK_ROWS, _CHUNK_ROWS), :]
        m = t_raw > 0.001
        t = jnp.where(m, t_raw, o)
        mf = jnp.where(m, one, zero)

        d = jnp.abs(o - t)
        d2 = d * d
        rt = 1.0 / t
        ro = 1.0 / o
        lr = jnp.log(o * rt)
        lr2 = lr * lr
        alr = jnp.where(m, jnp.abs(lr), big)
        rdiff = ro - rt
        qs = (
            mf,
            d2,
            d,
            lr2,
            d * rt,
            d2 * rt,
            jnp.where(alr < _LN125, one, zero),
            jnp.where(alr < 2.0 * _LN125, one, zero),
            jnp.where(alr < 3.0 * _LN125, one, zero),
            rdiff * rdiff,
            jnp.abs(rdiff),
        )
        return tuple(c + _lane_reduce(q) for c, q in zip(carry, qs))

    init = tuple(jnp.zeros((_CHUNK_ROWS, 128), jnp.float32)
                 for _ in range(_NQ))
    acc = jax.lax.fori_loop(0, _BLOCK_ROWS // _CHUNK_ROWS, body, init,
                            unroll=2)

    @pl.when(step == 0)
    def _init():
        for q in range(_NQ):
            acc_ref[q] = acc[q]

    @pl.when(step != 0)
    def _accum():
        for q in range(_NQ):
            acc_ref[q] += acc[q]

    @pl.when(step == nsteps - 1)
    def _finalize():
        s = [jnp.sum(acc_ref[q]) for q in range(_NQ)]
        inv_count = 1.0 / s[0]
        out_ref[0] = jnp.sqrt(s[1] * inv_count)
        out_ref[1] = s[2] * inv_count
        out_ref[2] = s[4] * inv_count
        out_ref[3] = s[6] * inv_count
        out_ref[4] = s[7] * inv_count
        out_ref[5] = s[8] * inv_count
        out_ref[6] = 1000.0 * jnp.sqrt(s[9] * inv_count)
        out_ref[7] = 1000.0 * s[10] * inv_count
        out_ref[8] = s[5] * inv_count
        out_ref[9] = jnp.sqrt(s[3] * inv_count)


def kernel(outputs, target):
    o = outputs.reshape(_ROWS, _COLS)
    t = target.reshape(_ROWS, _COLS)
    grid = _ROWS // _BLOCK_ROWS
    res = pl.pallas_call(
        _metrics_kernel,
        grid=(grid,),
        in_specs=[
            pl.BlockSpec((_BLOCK_ROWS, _COLS), lambda i: (i, 0)),
            pl.BlockSpec((_BLOCK_ROWS, _COLS), lambda i: (i, 0)),
        ],
        out_specs=pl.BlockSpec(memory_space=pltpu.SMEM),
        out_shape=jax.ShapeDtypeStruct((10,), jnp.float32),
        scratch_shapes=[pltpu.VMEM((_NQ, _CHUNK_ROWS, 128), jnp.float32)],
    )(o, t)
    return (res[0], res[1], res[2], res[3], res[4], res[5], res[6], res[7],
            res[8], res[9])


# speedup vs baseline: 1.4091x; 1.0367x over previous
import math

import jax
import jax.numpy as jnp
from jax.experimental import pallas as pl
from jax.experimental.pallas import tpu as pltpu

_LN125 = math.log(1.25)

_ROWS = 2048
_COLS = 1024
_BLOCK_ROWS = 512
_CHUNK_ROWS = 8
_NQ = 11


def _lane_reduce(q):
    parts = [q[:, j * 128:(j + 1) * 128] for j in range(_COLS // 128)]
    while len(parts) > 1:
        parts = [a + b for a, b in zip(parts[::2], parts[1::2])]
    return parts[0]


def _metrics_kernel(o_ref, t_ref, out_ref, acc_ref):
    step = pl.program_id(0)
    nsteps = pl.num_programs(0)
    big = jnp.float32(1e30)
    one = jnp.float32(1.0)
    zero = jnp.float32(0.0)

    def body(i, carry):
        o = o_ref[pl.ds(i * _CHUNK_ROWS, _CHUNK_ROWS), :]
        t_raw = t_ref[pl.ds(i * _CHUNK_ROWS, _CHUNK_ROWS), :]
        m = t_raw > 0.001
        t = jnp.where(m, t_raw, o)
        mf = jnp.where(m, one, zero)

        d = jnp.abs(o - t)
        d2 = d * d
        rt = 1.0 / t
        ro = 1.0 / o
        lr = jnp.log(o * rt)
        lr2 = lr * lr
        alr = jnp.where(m, jnp.abs(lr), big)
        rdiff = ro - rt
        qs = (
            mf,
            d2,
            d,
            lr2,
            d * rt,
            d2 * rt,
            jnp.where(alr < _LN125, one, zero),
            jnp.where(alr < 2.0 * _LN125, one, zero),
            jnp.where(alr < 3.0 * _LN125, one, zero),
            rdiff * rdiff,
            jnp.abs(rdiff),
        )
        return tuple(c + _lane_reduce(q) for c, q in zip(carry, qs))

    init = tuple(jnp.zeros((_CHUNK_ROWS, 128), jnp.float32)
                 for _ in range(_NQ))
    acc = jax.lax.fori_loop(0, _BLOCK_ROWS // _CHUNK_ROWS, body, init,
                            unroll=16)

    @pl.when(step == 0)
    def _init():
        for q in range(_NQ):
            acc_ref[q] = acc[q]

    @pl.when(step != 0)
    def _accum():
        for q in range(_NQ):
            acc_ref[q] += acc[q]

    @pl.when(step == nsteps - 1)
    def _finalize():
        s = [jnp.sum(acc_ref[q]) for q in range(_NQ)]
        inv_count = 1.0 / s[0]
        out_ref[0] = jnp.sqrt(s[1] * inv_count)
        out_ref[1] = s[2] * inv_count
        out_ref[2] = s[4] * inv_count
        out_ref[3] = s[6] * inv_count
        out_ref[4] = s[7] * inv_count
        out_ref[5] = s[8] * inv_count
        out_ref[6] = 1000.0 * jnp.sqrt(s[9] * inv_count)
        out_ref[7] = 1000.0 * s[10] * inv_count
        out_ref[8] = s[5] * inv_count
        out_ref[9] = jnp.sqrt(s[3] * inv_count)


def kernel(outputs, target):
    o = outputs.reshape(_ROWS, _COLS)
    t = target.reshape(_ROWS, _COLS)
    grid = _ROWS // _BLOCK_ROWS
    res = pl.pallas_call(
        _metrics_kernel,
        grid=(grid,),
        in_specs=[
            pl.BlockSpec((_BLOCK_ROWS, _COLS), lambda i: (i, 0)),
            pl.BlockSpec((_BLOCK_ROWS, _COLS), lambda i: (i, 0)),
        ],
        out_specs=pl.BlockSpec(memory_space=pltpu.SMEM),
        out_shape=jax.ShapeDtypeStruct((10,), jnp.float32),
        scratch_shapes=[pltpu.VMEM((_NQ, _CHUNK_ROWS, 128), jnp.float32)],
    )(o, t)
    return (res[0], res[1], res[2], res[3], res[4], res[5], res[6], res[7],
            res[8], res[9])
